# baseline (speedup 1.0000x reference)
import functools

import jax
import jax.numpy as jnp
from jax import lax
from jax.experimental import pallas as pl
from jax.experimental.pallas import tpu as pltpu
from jax.experimental.pallas import tpu_sc as plsc

NUM_EMB = 1_000_000
DIM = 64
PDIM = 128
ROWS = 16384
COLS = 26
B = ROWS * COLS

NC = 2
NS = 16
NW = NC * NS

IDX_W = 128
CHUNK = 256
SUB = CHUNK // IDX_W
B_PER_W = B // NW
N_CHUNKS = B_PER_W // CHUNK
IDX_ROWS = B_PER_W // IDX_W

assert B_PER_W % CHUNK == 0 and CHUNK % IDX_W == 0 and N_CHUNKS % 2 == 0

_mesh = plsc.VectorSubcoreMesh(core_axis_name="c", subcore_axis_name="s")


@functools.partial(
    pl.kernel,
    mesh=_mesh,
    out_type=jax.ShapeDtypeStruct((B, PDIM), jnp.float32),
    scratch_types=[
        pltpu.VMEM((IDX_ROWS, IDX_W), jnp.int32),
        pltpu.VMEM((CHUNK, PDIM), jnp.float32),
        pltpu.VMEM((CHUNK, PDIM), jnp.float32),
        pltpu.SemaphoreType.DMA,
        pltpu.SemaphoreType.DMA,
        pltpu.SemaphoreType.DMA,
    ],
)
def _emb_lookup(idx_hbm, table_hbm, out_hbm, idx_v, rows0, rows1, gsem,
                osem0, osem1):
    wid = lax.axis_index("s") * NC + lax.axis_index("c")
    row0 = wid * IDX_ROWS
    base = wid * B_PER_W

    def gather(i, rbuf):
        for j in range(SUB):
            pltpu.async_copy(
                table_hbm.at[idx_v.at[i * SUB + j]],
                rbuf.at[pl.ds(j * IDX_W, IDX_W)],
                gsem,
            )

    def wait_gather(rbuf):
        for j in range(SUB):
            pltpu.make_async_copy(
                table_hbm.at[idx_v.at[j]],
                rbuf.at[pl.ds(j * IDX_W, IDX_W)],
                gsem,
            ).wait()

    def store(i, rbuf, osem):
        pltpu.async_copy(
            rbuf, out_hbm.at[pl.ds(base + i * CHUNK, CHUNK)], osem)

    def wait_store(rbuf, osem):
        pltpu.make_async_copy(
            rbuf, out_hbm.at[pl.ds(base, CHUNK)], osem).wait()

    pltpu.sync_copy(idx_hbm.at[pl.ds(row0, IDX_ROWS)], idx_v)

    gather(0, rows0)
    wait_gather(rows0)
    gather(1, rows1)
    store(0, rows0, osem0)

    def body(k, _):
        i1 = 2 * k + 1
        wait_gather(rows1)
        wait_store(rows0, osem0)
        gather(i1 + 1, rows0)
        store(i1, rows1, osem1)
        i2 = 2 * k + 2
        wait_gather(rows0)
        wait_store(rows1, osem1)
        gather(i2 + 1, rows1)
        store(i2, rows0, osem0)
        return 0

    lax.fori_loop(0, N_CHUNKS // 2 - 1, body, 0)

    last = N_CHUNKS - 1
    wait_gather(rows1)
    wait_store(rows0, osem0)
    store(last, rows1, osem1)
    wait_store(rows1, osem1)


def kernel(x, Weights):
    idx = x.reshape(B // IDX_W, IDX_W).astype(jnp.int32)
    table = jnp.pad(Weights, ((0, 0), (0, PDIM - DIM)))
    out = _emb_lookup(idx, table)
    return out[:, :DIM].reshape(ROWS, COLS, DIM)


# speedup vs baseline: 1.2586x; 1.2586x over previous
import functools

import jax
import jax.numpy as jnp
from jax import lax
from jax.experimental import pallas as pl
from jax.experimental.pallas import tpu as pltpu
from jax.experimental.pallas import tpu_sc as plsc

NUM_EMB = 1_000_000
DIM = 64
PDIM = 128
ROWS = 16384
COLS = 26
SLOTS = 32

NC = 2
NS = 16
NW = NC * NS

CB = 8
R_PER_W = ROWS // NW
N_CHUNKS = R_PER_W // CB
CROWS = CB * SLOTS

assert R_PER_W % CB == 0 and N_CHUNKS % 2 == 0

_mesh = plsc.VectorSubcoreMesh(core_axis_name="c", subcore_axis_name="s")


@functools.partial(
    pl.kernel,
    mesh=_mesh,
    out_type=jax.ShapeDtypeStruct((ROWS * SLOTS, PDIM), jnp.float32),
    scratch_types=[
        pltpu.VMEM((R_PER_W, COLS), jnp.int32),
        pltpu.VMEM((CROWS, PDIM), jnp.float32),
        pltpu.VMEM((CROWS, PDIM), jnp.float32),
        pltpu.SemaphoreType.DMA,
        pltpu.SemaphoreType.DMA,
        pltpu.SemaphoreType.DMA,
    ],
)
def _emb_lookup(idx_hbm, table_hbm, out_hbm, idx_v, rows0, rows1, gsem,
                osem0, osem1):
    wid = lax.axis_index("s") * NC + lax.axis_index("c")
    row0 = wid * R_PER_W

    def gather(i, rbuf):
        for blk in range(CB):
            pltpu.async_copy(
                table_hbm.at[idx_v.at[i * CB + blk]],
                rbuf.at[pl.ds(blk * SLOTS, COLS)],
                gsem,
            )

    def wait_gather(rbuf):
        for blk in range(CB):
            pltpu.make_async_copy(
                table_hbm.at[idx_v.at[blk]],
                rbuf.at[pl.ds(blk * SLOTS, COLS)],
                gsem,
            ).wait()

    def store(i, rbuf, osem):
        pltpu.async_copy(
            rbuf, out_hbm.at[pl.ds((row0 + i * CB) * SLOTS, CROWS)], osem)

    def wait_store(rbuf, osem):
        pltpu.make_async_copy(
            rbuf, out_hbm.at[pl.ds(0, CROWS)], osem).wait()

    pltpu.sync_copy(idx_hbm.at[pl.ds(row0, R_PER_W)], idx_v)

    gather(0, rows0)
    wait_gather(rows0)
    gather(1, rows1)
    store(0, rows0, osem0)

    def body(k, _):
        i1 = 2 * k + 1
        wait_gather(rows1)
        wait_store(rows0, osem0)
        gather(i1 + 1, rows0)
        store(i1, rows1, osem1)
        i2 = 2 * k + 2
        wait_gather(rows0)
        wait_store(rows1, osem1)
        gather(i2 + 1, rows1)
        store(i2, rows0, osem0)
        return 0

    lax.fori_loop(0, N_CHUNKS // 2 - 1, body, 0)

    wait_gather(rows1)
    wait_store(rows0, osem0)
    store(N_CHUNKS - 1, rows1, osem1)
    wait_store(rows1, osem1)


def kernel(x, Weights):
    idx = x.astype(jnp.int32)
    table = jnp.pad(Weights, ((0, 0), (0, PDIM - DIM)))
    out = _emb_lookup(idx, table)
    return out.reshape(ROWS, SLOTS, PDIM)[:, :COLS, :DIM]
